# Initial kernel scaffold; baseline (speedup 1.0000x reference)
import functools

import jax
import jax.numpy as jnp
from jax import lax
from jax.experimental import pallas as pl
from jax.experimental.pallas import tpu as pltpu
from jax.experimental.pallas import tpu_sc as plsc

N = 10000
D = 128
HP = 16
E = 320000
NC, NS = 2, 16
NW = NC * NS
EPW = 10240
EP = NW * EPW
CH = 128
NCH = EPW // CH
RPT = N // NS

_SELU_SCALE = 1.0507009873554805
_SELU_ALPHA = 1.6732632423543772


def _selu(x):
    return _SELU_SCALE * jnp.where(x > 0, x, _SELU_ALPHA * jnp.expm1(x))


_BLK = 1000


def _mm_body(x_ref, w_ref, o_ref):
    o_ref[...] = jnp.dot(x_ref[...], w_ref[...],
                         preferred_element_type=jnp.float32)


def _mm(x, w):
    K = x.shape[1]
    F = w.shape[1]
    return pl.pallas_call(
        _mm_body,
        grid=(N // _BLK,),
        in_specs=[
            pl.BlockSpec((_BLK, K), lambda i: (i, 0)),
            pl.BlockSpec((K, F), lambda i: (0, 0)),
        ],
        out_specs=pl.BlockSpec((_BLK, F), lambda i: (i, 0)),
        out_shape=jax.ShapeDtypeStruct((N, F), jnp.float32),
    )(x, w)


def _act_mm_body(p_ref, b_ref, w_ref, o_ref):
    h = _selu(p_ref[0] + p_ref[1]) + b_ref[...]
    o_ref[...] = jnp.dot(h, w_ref[...], preferred_element_type=jnp.float32)


def _act_mm(parts, b, w):
    F = w.shape[1]
    return pl.pallas_call(
        _act_mm_body,
        grid=(N // _BLK,),
        in_specs=[
            pl.BlockSpec((2, _BLK, HP), lambda i: (0, i, 0)),
            pl.BlockSpec((1, HP), lambda i: (0, 0)),
            pl.BlockSpec((HP, F), lambda i: (0, 0)),
        ],
        out_specs=pl.BlockSpec((_BLK, F), lambda i: (i, 0)),
        out_shape=jax.ShapeDtypeStruct((N, F), jnp.float32),
    )(parts, b, w)


def _heads_body(p_ref, b_ref, w_ref, bc_ref, o_ref):
    h = _selu(p_ref[0] + p_ref[1]) + b_ref[...]
    o_ref[...] = jnp.dot(h, w_ref[...],
                         preferred_element_type=jnp.float32) + bc_ref[...]


def _heads(parts, b, wcat, bcat):
    return pl.pallas_call(
        _heads_body,
        grid=(N // _BLK,),
        in_specs=[
            pl.BlockSpec((2, _BLK, HP), lambda i: (0, i, 0)),
            pl.BlockSpec((1, HP), lambda i: (0, 0)),
            pl.BlockSpec((HP, 128), lambda i: (0, 0)),
            pl.BlockSpec((1, 128), lambda i: (0, 0)),
        ],
        out_specs=pl.BlockSpec((_BLK, 128), lambda i: (i, 0)),
        out_shape=jax.ShapeDtypeStruct((N, 128), jnp.float32),
    )(parts, b, wcat, bcat)


def _spmm_body(sup_hbm, src_hbm, dst_hbm, vals_hbm, zero_hbm, out_hbm,
               accum, src_v, dst_v, vals_v, rows_v, sem):
    cid = lax.axis_index("c")
    sid = lax.axis_index("s")
    wid = cid * NS + sid
    r0 = sid * RPT
    pltpu.sync_copy(zero_hbm, accum.at[pl.ds(r0, RPT)])
    plsc.subcore_barrier()

    def chunk_body(c, _):
        base = wid * EPW + c * CH
        pltpu.sync_copy(src_hbm.at[pl.ds(base, CH)], src_v)
        pltpu.sync_copy(dst_hbm.at[pl.ds(base, CH)], dst_v)
        pltpu.sync_copy(vals_hbm.at[pl.ds(base, CH)], vals_v)
        pltpu.async_copy(sup_hbm.at[src_v], rows_v, sem).wait()

        def e_body(e, _):
            idx = jnp.full((16,), e, jnp.int32)
            val = plsc.load_gather(vals_v, [idx])
            rows_v[e] = rows_v[e] * val
            return 0

        lax.fori_loop(0, CH, e_body, 0)
        pltpu.sync_copy(rows_v, accum.at[dst_v], add=True)
        return 0

    lax.fori_loop(0, NCH, chunk_body, 0)
    plsc.subcore_barrier()
    pltpu.sync_copy(accum.at[pl.ds(r0, RPT)],
                    out_hbm.at[pl.ds(cid * N + r0, RPT)])


_spmm = pl.kernel(
    _spmm_body,
    out_type=jax.ShapeDtypeStruct((NC * N, HP), jnp.float32),
    mesh=plsc.VectorSubcoreMesh(core_axis_name="c", subcore_axis_name="s"),
    scratch_types=[
        pltpu.VMEM_SHARED((N, HP), jnp.float32),
        pltpu.VMEM((CH,), jnp.int32),
        pltpu.VMEM((CH,), jnp.int32),
        pltpu.VMEM((CH,), jnp.float32),
        pltpu.VMEM((CH, HP), jnp.float32),
        pltpu.SemaphoreType.DMA,
    ],
)


def kernel(x, adj_indices, adj_values, W1, b1, W2, b2, W3, b3,
           Wc0, bc0, Wc1, bc1, Wc2, bc2):
    dst = adj_indices[0].astype(jnp.int32)
    src = adj_indices[1].astype(jnp.int32)
    pad = EP - E
    srcp = jnp.concatenate([src, jnp.zeros((pad,), jnp.int32)])
    dstp = jnp.concatenate([dst, jnp.zeros((pad,), jnp.int32)])
    valsp = jnp.concatenate([adj_values, jnp.zeros((pad,), jnp.float32)])
    zero = jnp.zeros((RPT, HP), jnp.float32)

    W1p = jnp.zeros((D, HP), jnp.float32).at[:, :11].set(W1)
    W2p = jnp.zeros((HP, HP), jnp.float32).at[:11, :11].set(W2)
    W3p = jnp.zeros((HP, HP), jnp.float32).at[:11, :11].set(W3)
    b1p = jnp.zeros((1, HP), jnp.float32).at[0, :11].set(b1)
    b2p = jnp.zeros((1, HP), jnp.float32).at[0, :11].set(b2)
    b3p = jnp.zeros((1, HP), jnp.float32).at[0, :11].set(b3)
    wcat = jnp.zeros((HP, 128), jnp.float32)
    wcat = wcat.at[:11, 0:8].set(Wc0)
    wcat = wcat.at[:11, 8:24].set(Wc1)
    wcat = wcat.at[:11, 24:28].set(Wc2)
    bcat = jnp.zeros((1, 128), jnp.float32)
    bcat = bcat.at[0, 0:8].set(bc0)
    bcat = bcat.at[0, 8:24].set(bc1)
    bcat = bcat.at[0, 24:28].set(bc2)

    sup = _mm(x, W1p)
    parts = _spmm(sup, srcp, dstp, valsp, zero).reshape(NC, N, HP)
    sup = _act_mm(parts, b1p, W2p)
    parts = _spmm(sup, srcp, dstp, valsp, zero).reshape(NC, N, HP)
    sup = _act_mm(parts, b2p, W3p)
    parts = _spmm(sup, srcp, dstp, valsp, zero).reshape(NC, N, HP)
    outc = _heads(parts, b3p, wcat, bcat)
    return (outc[:, 0:8], outc[:, 8:24], outc[:, 24:28])


# speedup vs baseline: 6.6519x; 6.6519x over previous
import functools

import jax
import jax.numpy as jnp
from jax import lax
from jax.experimental import pallas as pl
from jax.experimental.pallas import tpu as pltpu
from jax.experimental.pallas import tpu_sc as plsc

N = 10000
D = 128
HP = 16
E = 320000
NC, NS = 2, 16
NW = NC * NS
EPW = 10240
EP = NW * EPW
CH = 128
NCH = EPW // CH
NP = 10240
RPT = NP // NS

_SELU_SCALE = 1.0507009873554805
_SELU_ALPHA = 1.6732632423543772


def _selu(x):
    return _SELU_SCALE * jnp.where(x > 0, x, _SELU_ALPHA * (jnp.exp(x) - 1.0))


_BLK = 1000


def _mm_body(x_ref, w_ref, o_ref):
    o_ref[...] = jnp.dot(x_ref[...], w_ref[...],
                         preferred_element_type=jnp.float32)


def _mm(x, w):
    K = x.shape[1]
    F = w.shape[1]
    return pl.pallas_call(
        _mm_body,
        grid=(N // _BLK,),
        in_specs=[
            pl.BlockSpec((_BLK, K), lambda i: (i, 0)),
            pl.BlockSpec((K, F), lambda i: (0, 0)),
        ],
        out_specs=pl.BlockSpec((_BLK, F), lambda i: (i, 0)),
        out_shape=jax.ShapeDtypeStruct((N, F), jnp.float32),
    )(x, w)


def _act_mm_body(p_ref, b_ref, w_ref, o_ref):
    h = _selu(p_ref[0] + p_ref[1]) + b_ref[...]
    o_ref[...] = jnp.dot(h, w_ref[...], preferred_element_type=jnp.float32)


def _act_mm(parts, b, w):
    F = w.shape[1]
    return pl.pallas_call(
        _act_mm_body,
        grid=(N // _BLK,),
        in_specs=[
            pl.BlockSpec((2, _BLK, HP), lambda i: (0, i, 0)),
            pl.BlockSpec((1, HP), lambda i: (0, 0)),
            pl.BlockSpec((HP, F), lambda i: (0, 0)),
        ],
        out_specs=pl.BlockSpec((_BLK, F), lambda i: (i, 0)),
        out_shape=jax.ShapeDtypeStruct((N, F), jnp.float32),
    )(parts, b, w)


def _heads_body(p_ref, b_ref, w_ref, bc_ref, o_ref):
    h = _selu(p_ref[0] + p_ref[1]) + b_ref[...]
    o_ref[...] = jnp.dot(h, w_ref[...],
                         preferred_element_type=jnp.float32) + bc_ref[...]


def _heads(parts, b, wcat, bcat):
    return pl.pallas_call(
        _heads_body,
        grid=(N // _BLK,),
        in_specs=[
            pl.BlockSpec((2, _BLK, HP), lambda i: (0, i, 0)),
            pl.BlockSpec((1, HP), lambda i: (0, 0)),
            pl.BlockSpec((HP, 128), lambda i: (0, 0)),
            pl.BlockSpec((1, 128), lambda i: (0, 0)),
        ],
        out_specs=pl.BlockSpec((_BLK, 128), lambda i: (i, 0)),
        out_shape=jax.ShapeDtypeStruct((N, 128), jnp.float32),
    )(parts, b, wcat, bcat)


def _spmm_body(sup_hbm, src_hbm, dst_hbm, vals_hbm, zero_hbm, out_hbm,
               accum, src_v, dst_v, vals_v, rows_v, sem):
    cid = lax.axis_index("c")
    sid = lax.axis_index("s")
    wid = cid * NS + sid
    r0 = sid * RPT
    pltpu.sync_copy(zero_hbm, accum.at[pl.ds(r0, RPT)])
    plsc.subcore_barrier()

    def chunk_body(c, _):
        base = wid * EPW + c * CH
        pltpu.sync_copy(src_hbm.at[pl.ds(base, CH)], src_v)
        pltpu.sync_copy(dst_hbm.at[pl.ds(base, CH)], dst_v)
        pltpu.sync_copy(vals_hbm.at[pl.ds(base, CH)], vals_v)
        pltpu.async_copy(sup_hbm.at[src_v], rows_v, sem).wait()

        for g in range(CH // 16):
            vals16 = vals_v[pl.ds(g * 16, 16)]
            for j in range(16):
                e = g * 16 + j
                rows_v[e] = rows_v[e] * vals16[j]
        pltpu.sync_copy(rows_v, accum.at[dst_v], add=True)
        return 0

    lax.fori_loop(0, NCH, chunk_body, 0)
    plsc.subcore_barrier()
    pltpu.sync_copy(accum.at[pl.ds(r0, RPT)],
                    out_hbm.at[pl.ds(cid * NP + r0, RPT)])


_spmm = pl.kernel(
    _spmm_body,
    out_type=jax.ShapeDtypeStruct((NC * NP, HP), jnp.float32),
    mesh=plsc.VectorSubcoreMesh(core_axis_name="c", subcore_axis_name="s"),
    compiler_params=pltpu.CompilerParams(use_tc_tiling_on_sc=False),
    scratch_types=[
        pltpu.VMEM_SHARED((NP, HP), jnp.float32),
        pltpu.VMEM((CH,), jnp.int32),
        pltpu.VMEM((CH,), jnp.int32),
        pltpu.VMEM((CH,), jnp.float32),
        pltpu.VMEM((CH, HP), jnp.float32),
        pltpu.SemaphoreType.DMA,
    ],
)


def kernel(x, adj_indices, adj_values, W1, b1, W2, b2, W3, b3,
           Wc0, bc0, Wc1, bc1, Wc2, bc2):
    dst = adj_indices[0].astype(jnp.int32)
    src = adj_indices[1].astype(jnp.int32)
    pad = EP - E
    srcp = jnp.concatenate([src, jnp.zeros((pad,), jnp.int32)])
    dstp = jnp.concatenate([dst, jnp.zeros((pad,), jnp.int32)])
    valsp = jnp.concatenate([adj_values, jnp.zeros((pad,), jnp.float32)])
    zero = jnp.zeros((RPT, HP), jnp.float32)

    W1p = jnp.zeros((D, HP), jnp.float32).at[:, :11].set(W1)
    W2p = jnp.zeros((HP, HP), jnp.float32).at[:11, :11].set(W2)
    W3p = jnp.zeros((HP, HP), jnp.float32).at[:11, :11].set(W3)
    b1p = jnp.zeros((1, HP), jnp.float32).at[0, :11].set(b1)
    b2p = jnp.zeros((1, HP), jnp.float32).at[0, :11].set(b2)
    b3p = jnp.zeros((1, HP), jnp.float32).at[0, :11].set(b3)
    wcat = jnp.zeros((HP, 128), jnp.float32)
    wcat = wcat.at[:11, 0:8].set(Wc0)
    wcat = wcat.at[:11, 8:24].set(Wc1)
    wcat = wcat.at[:11, 24:28].set(Wc2)
    bcat = jnp.zeros((1, 128), jnp.float32)
    bcat = bcat.at[0, 0:8].set(bc0)
    bcat = bcat.at[0, 8:24].set(bc1)
    bcat = bcat.at[0, 24:28].set(bc2)

    sup = _mm(x, W1p)
    parts = _spmm(sup, srcp, dstp, valsp, zero).reshape(NC, NP, HP)
    sup = _act_mm(parts, b1p, W2p)
    parts = _spmm(sup, srcp, dstp, valsp, zero).reshape(NC, NP, HP)
    sup = _act_mm(parts, b2p, W3p)
    parts = _spmm(sup, srcp, dstp, valsp, zero).reshape(NC, NP, HP)
    outc = _heads(parts, b3p, wcat, bcat)
    return (outc[:, 0:8], outc[:, 8:24], outc[:, 24:28])


# speedup vs baseline: 15.8607x; 2.3844x over previous
import functools

import jax
import jax.numpy as jnp
from jax import lax
from jax.experimental import pallas as pl
from jax.experimental.pallas import tpu as pltpu
from jax.experimental.pallas import tpu_sc as plsc

N = 10000
D = 128
HP = 16
E = 320000
NC, NS = 2, 16
NW = NC * NS
EPW = 10240
EP = NW * EPW
CH = 128
NCH = EPW // CH
NP = 10240
RPT = NP // NS

_SELU_SCALE = 1.0507009873554805
_SELU_ALPHA = 1.6732632423543772


def _selu(x):
    return _SELU_SCALE * jnp.where(x > 0, x, _SELU_ALPHA * (jnp.exp(x) - 1.0))


_BLK = 1000


def _mm_body(x_ref, w_ref, o_ref):
    o_ref[...] = jnp.dot(x_ref[...], w_ref[...],
                         preferred_element_type=jnp.float32)


def _mm(x, w):
    K = x.shape[1]
    F = w.shape[1]
    return pl.pallas_call(
        _mm_body,
        grid=(N // _BLK,),
        in_specs=[
            pl.BlockSpec((_BLK, K), lambda i: (i, 0)),
            pl.BlockSpec((K, F), lambda i: (0, 0)),
        ],
        out_specs=pl.BlockSpec((_BLK, F), lambda i: (i, 0)),
        out_shape=jax.ShapeDtypeStruct((N, F), jnp.float32),
    )(x, w)


def _act_mm_body(p_ref, b_ref, w_ref, o_ref):
    h = _selu(p_ref[0] + p_ref[1]) + b_ref[...]
    o_ref[...] = jnp.dot(h, w_ref[...], preferred_element_type=jnp.float32)


def _act_mm(parts, b, w):
    F = w.shape[1]
    return pl.pallas_call(
        _act_mm_body,
        grid=(N // _BLK,),
        in_specs=[
            pl.BlockSpec((2, _BLK, HP), lambda i: (0, i, 0)),
            pl.BlockSpec((1, HP), lambda i: (0, 0)),
            pl.BlockSpec((HP, F), lambda i: (0, 0)),
        ],
        out_specs=pl.BlockSpec((_BLK, F), lambda i: (i, 0)),
        out_shape=jax.ShapeDtypeStruct((N, F), jnp.float32),
    )(parts, b, w)


def _heads_body(p_ref, b_ref, w_ref, bc_ref, o_ref):
    h = _selu(p_ref[0] + p_ref[1]) + b_ref[...]
    o_ref[...] = jnp.dot(h, w_ref[...],
                         preferred_element_type=jnp.float32) + bc_ref[...]


def _heads(parts, b, wcat, bcat):
    return pl.pallas_call(
        _heads_body,
        grid=(N // _BLK,),
        in_specs=[
            pl.BlockSpec((2, _BLK, HP), lambda i: (0, i, 0)),
            pl.BlockSpec((1, HP), lambda i: (0, 0)),
            pl.BlockSpec((HP, 128), lambda i: (0, 0)),
            pl.BlockSpec((1, 128), lambda i: (0, 0)),
        ],
        out_specs=pl.BlockSpec((_BLK, 128), lambda i: (i, 0)),
        out_shape=jax.ShapeDtypeStruct((N, 128), jnp.float32),
    )(parts, b, wcat, bcat)


_NBUF = 4


def _spmm_body(sup_hbm, src_hbm, dst_hbm, vals_hbm, zero_hbm, out_hbm,
               accum, src_all, dst_all, vals_all,
               rows0, rows1, rows2, rows3, sem0, sem1, sem2, sem3):
    rows = [rows0, rows1, rows2, rows3]
    sems = [sem0, sem1, sem2, sem3]
    cid = lax.axis_index("c")
    sid = lax.axis_index("s")
    wid = cid * NS + sid
    r0 = sid * RPT
    pltpu.sync_copy(zero_hbm, accum.at[pl.ds(r0, RPT)])
    pltpu.sync_copy(src_hbm.at[wid], src_all)
    pltpu.sync_copy(dst_hbm.at[wid], dst_all)
    pltpu.sync_copy(vals_hbm.at[wid], vals_all)
    plsc.subcore_barrier()

    for b in range(_NBUF):
        pltpu.async_copy(sup_hbm.at[src_all.at[b]], rows[b], sems[b])

    def it_body(it, _):
        for b in range(_NBUF):
            c = it * _NBUF + b
            pltpu.make_async_copy(
                sup_hbm.at[src_all.at[c]], rows[b], sems[b]).wait()
            for g in range(CH // 16):
                vals16 = vals_all[pl.ds(c * CH + g * 16, 16)]
                for j in range(16):
                    e = g * 16 + j
                    rows[b][e] = rows[b][e] * vals16[j]
            pltpu.sync_copy(rows[b], accum.at[dst_all.at[c]], add=True)
            c2 = c + _NBUF

            @pl.when(c2 < NCH)
            def _():
                pltpu.async_copy(sup_hbm.at[src_all.at[c2]], rows[b], sems[b])
        return 0

    lax.fori_loop(0, NCH // _NBUF, it_body, 0)
    plsc.subcore_barrier()
    pltpu.sync_copy(accum.at[pl.ds(r0, RPT)],
                    out_hbm.at[pl.ds(cid * NP + r0, RPT)])


_spmm = pl.kernel(
    _spmm_body,
    out_type=jax.ShapeDtypeStruct((NC * NP, HP), jnp.float32),
    mesh=plsc.VectorSubcoreMesh(core_axis_name="c", subcore_axis_name="s"),
    compiler_params=pltpu.CompilerParams(use_tc_tiling_on_sc=False),
    scratch_types=[
        pltpu.VMEM_SHARED((NP, HP), jnp.float32),
        pltpu.VMEM((NCH, CH), jnp.int32),
        pltpu.VMEM((NCH, CH), jnp.int32),
        pltpu.VMEM((EPW,), jnp.float32),
        pltpu.VMEM((CH, HP), jnp.float32),
        pltpu.VMEM((CH, HP), jnp.float32),
        pltpu.VMEM((CH, HP), jnp.float32),
        pltpu.VMEM((CH, HP), jnp.float32),
        pltpu.SemaphoreType.DMA,
        pltpu.SemaphoreType.DMA,
        pltpu.SemaphoreType.DMA,
        pltpu.SemaphoreType.DMA,
    ],
)


def kernel(x, adj_indices, adj_values, W1, b1, W2, b2, W3, b3,
           Wc0, bc0, Wc1, bc1, Wc2, bc2):
    dst = adj_indices[0].astype(jnp.int32)
    src = adj_indices[1].astype(jnp.int32)
    pad = EP - E
    srcp = jnp.concatenate([src, jnp.zeros((pad,), jnp.int32)])
    dstp = jnp.concatenate([dst, jnp.zeros((pad,), jnp.int32)])
    valsp = jnp.concatenate([adj_values, jnp.zeros((pad,), jnp.float32)])
    srcp = srcp.reshape(NW, NCH, CH)
    dstp = dstp.reshape(NW, NCH, CH)
    valsp = valsp.reshape(NW, EPW)
    zero = jnp.zeros((RPT, HP), jnp.float32)

    W1p = jnp.zeros((D, HP), jnp.float32).at[:, :11].set(W1)
    W2p = jnp.zeros((HP, HP), jnp.float32).at[:11, :11].set(W2)
    W3p = jnp.zeros((HP, HP), jnp.float32).at[:11, :11].set(W3)
    b1p = jnp.zeros((1, HP), jnp.float32).at[0, :11].set(b1)
    b2p = jnp.zeros((1, HP), jnp.float32).at[0, :11].set(b2)
    b3p = jnp.zeros((1, HP), jnp.float32).at[0, :11].set(b3)
    wcat = jnp.zeros((HP, 128), jnp.float32)
    wcat = wcat.at[:11, 0:8].set(Wc0)
    wcat = wcat.at[:11, 8:24].set(Wc1)
    wcat = wcat.at[:11, 24:28].set(Wc2)
    bcat = jnp.zeros((1, 128), jnp.float32)
    bcat = bcat.at[0, 0:8].set(bc0)
    bcat = bcat.at[0, 8:24].set(bc1)
    bcat = bcat.at[0, 24:28].set(bc2)

    sup = _mm(x, W1p)
    parts = _spmm(sup, srcp, dstp, valsp, zero).reshape(NC, NP, HP)
    sup = _act_mm(parts, b1p, W2p)
    parts = _spmm(sup, srcp, dstp, valsp, zero).reshape(NC, NP, HP)
    sup = _act_mm(parts, b2p, W3p)
    parts = _spmm(sup, srcp, dstp, valsp, zero).reshape(NC, NP, HP)
    outc = _heads(parts, b3p, wcat, bcat)
    return (outc[:, 0:8], outc[:, 8:24], outc[:, 24:28])
